# baseline (speedup 1.0000x reference)
import jax
import jax.numpy as jnp
from jax import lax
from jax.experimental import pallas as pl
from jax.experimental.pallas import tpu as pltpu
from jax.experimental.pallas import tpu_sc as plsc

N = 10000
E = 320000
IN_FEATS = 128
N_HIDDEN = 64
C = 16
ALPHA = 0.1
NITER = 10

NS = 16
NP = 10240
RPT = NP // NS
EP = 327680
EPT = EP // NS
CH = 1024
NCHUNK = EPT // CH

HC = C // 2
NPH = NP // 2
RPT2 = NPH // NS

_MESH1 = dict(
    mesh=plsc.VectorSubcoreMesh(
        core_axis_name="c", subcore_axis_name="s", num_cores=1, num_subcores=NS
    ),
    compiler_params=pltpu.CompilerParams(
        use_tc_tiling_on_sc=False, needs_layout_passes=False),
)
_MESH2 = dict(
    mesh=plsc.VectorSubcoreMesh(
        core_axis_name="c", subcore_axis_name="s", num_cores=2, num_subcores=NS
    ),
    compiler_params=pltpu.CompilerParams(
        use_tc_tiling_on_sc=False, needs_layout_passes=False),
)


_BLK = 1024


def _dense_body(x_ref, w1_ref, w2_ref, loc_ref):
    h = jnp.tanh(jnp.dot(x_ref[...], w1_ref[...],
                         preferred_element_type=jnp.float32))
    loc_ref[...] = jnp.dot(h, w2_ref[...], preferred_element_type=jnp.float32)


def _dense_stage(x, w1, w2):
    return pl.pallas_call(
        _dense_body,
        grid=(NP // _BLK,),
        in_specs=[
            pl.BlockSpec((_BLK, IN_FEATS), lambda i: (i, 0)),
            pl.BlockSpec((IN_FEATS, N_HIDDEN), lambda i: (0, 0)),
            pl.BlockSpec((N_HIDDEN, C), lambda i: (0, 0)),
        ],
        out_specs=pl.BlockSpec((_BLK, C), lambda i: (i, 0)),
        out_shape=jax.ShapeDtypeStruct((NP, C), jnp.float32),
    )(x, w1, w2)


NBUF = 4


def _prop_body(loc2_hbm, col2_hbm, row2_hbm, ones_hbm, zeros_hbm,
               out_hbm,
               q_sh, acc_sh, cidx_a, ridx_a, msg0, msg1, msg2, msg3,
               a_v, b_v, sa_v, sb_v, sal_v, sbl_v,
               gs0, gs1, gs2, gs3, ss0, ss1, ss2, ss3):
    cix = lax.axis_index("c")
    w = lax.axis_index("s")
    rsl = pl.ds(pl.multiple_of(w * RPT, RPT), RPT)
    pltpu.sync_copy(col2_hbm.at[pl.ds(w * NCHUNK, NCHUNK)], cidx_a)
    pltpu.sync_copy(row2_hbm.at[pl.ds(w * NCHUNK, NCHUNK)], ridx_a)
    lane = lax.iota(jnp.int32, 16)
    r0 = lax.shift_right_logical(lane, 3)
    c16 = lax.bitwise_and(lane, 7)

    pltpu.sync_copy(zeros_hbm, acc_sh.at[rsl])
    pltpu.sync_copy(ones_hbm, msg0)
    pltpu.sync_copy(loc2_hbm.at[cix, rsl], b_v)
    plsc.subcore_barrier()
    dcs = [pltpu.async_copy(msg0, acc_sh.at[ridx_a.at[j]], gs0, add=True)
           for j in range(NCHUNK)]
    for cp in dcs:
        cp.wait()
    plsc.subcore_barrier()

    pltpu.sync_copy(acc_sh.at[rsl], a_v)

    def prep(i, carry):
        ri = r0 + 2 * i
        d = plsc.load_gather(a_v, [ri, c16]) + 1.0
        lo = plsc.load_gather(b_v, [ri, c16])
        ii = 0x5F3759DF - lax.shift_right_logical(
            plsc.bitcast(d, jnp.int32), 1)
        y = plsc.bitcast(ii, jnp.float32)
        for _ in range(4):
            y = y * (1.5 - 0.5 * d * y * y)
        q0 = y * lo
        plsc.store_scatter(sa_v, [ri, c16], 0.9 * y * y)
        plsc.store_scatter(sal_v, [ri, c16], 0.9 * y)
        plsc.store_scatter(sb_v, [ri, c16], ALPHA * q0)
        plsc.store_scatter(sbl_v, [ri, c16], ALPHA * lo)
        plsc.store_scatter(a_v, [ri, c16], q0)
        return carry

    lax.fori_loop(0, RPT // 2, prep, 0)
    bufs = (q_sh, acc_sh)
    pltpu.sync_copy(a_v, q_sh.at[rsl])
    pltpu.sync_copy(a_v, acc_sh.at[rsl])
    plsc.subcore_barrier()

    msgs = (msg0, msg1, msg2, msg3)
    gsems = (gs0, gs1, gs2, gs3)
    ssems = (ss0, ss1, ss2, ss3)

    for t in range(NITER):
        S = bufs[t % 2]
        ACC = bufs[(t + 1) % 2]
        s1_v = sa_v if t < NITER - 1 else sal_v
        s2_v = sb_v if t < NITER - 1 else sbl_v

        gcp = [None] * NCHUNK
        scp = [None] * NCHUNK
        gcp[0] = pltpu.async_copy(S.at[cidx_a.at[0]], msgs[0], gsems[0])
        if NCHUNK > 1:
            gcp[1] = pltpu.async_copy(S.at[cidx_a.at[1]], msgs[1], gsems[1])
        for j in range(NCHUNK):
            b = j % NBUF
            gcp[j].wait()
            scp[j] = pltpu.async_copy(
                msgs[b], ACC.at[ridx_a.at[j]], ssems[b], add=True)
            nxt = j + 2
            if nxt < NCHUNK:
                nb = nxt % NBUF
                if nxt >= NBUF:
                    scp[nxt - NBUF].wait()
                gcp[nxt] = pltpu.async_copy(
                    S.at[cidx_a.at[nxt]], msgs[nb], gsems[nb])
        for j in range(max(0, NCHUNK - NBUF), NCHUNK):
            scp[j].wait()
        plsc.subcore_barrier()

        pltpu.sync_copy(ACC.at[rsl], a_v)

        def row(i, carry):
            ri = r0 + 2 * i
            a = plsc.load_gather(a_v, [ri, c16])
            bb = plsc.load_gather(s1_v, [ri, c16])
            cc = plsc.load_gather(s2_v, [ri, c16])
            plsc.store_scatter(a_v, [ri, c16], a * bb + cc)
            return carry

        lax.fori_loop(0, RPT // 2, row, 0)
        if t < NITER - 1:
            pltpu.sync_copy(a_v, ACC.at[rsl])
            pltpu.sync_copy(a_v, S.at[rsl])
        else:
            pltpu.sync_copy(a_v, out_hbm.at[cix, rsl])
        plsc.subcore_barrier()


def _build(interpret=False):
    prop_kernel = pl.kernel(
        _prop_body,
        out_type=jax.ShapeDtypeStruct((2, NP, HC), jnp.float32),
        scratch_types=[
            pltpu.VMEM_SHARED((NP, HC), jnp.float32),
            pltpu.VMEM_SHARED((NP, HC), jnp.float32),
            pltpu.VMEM((NCHUNK, CH), jnp.int32),
            pltpu.VMEM((NCHUNK, CH), jnp.int32),
            pltpu.VMEM((CH, HC), jnp.float32),
            pltpu.VMEM((CH, HC), jnp.float32),
            pltpu.VMEM((CH, HC), jnp.float32),
            pltpu.VMEM((CH, HC), jnp.float32),
            pltpu.VMEM((RPT, HC), jnp.float32),
            pltpu.VMEM((RPT, HC), jnp.float32),
            pltpu.VMEM((RPT, HC), jnp.float32),
            pltpu.VMEM((RPT, HC), jnp.float32),
            pltpu.VMEM((RPT, HC), jnp.float32),
            pltpu.VMEM((RPT, HC), jnp.float32),
        ] + [pltpu.SemaphoreType.DMA] * 8,
        interpret=interpret,
        **_MESH2,
    )
    return prop_kernel


_prop_kernel = _build()


def kernel(local_preds, edge_index, W1, W2):
    npad = EP - E
    row = jnp.concatenate(
        [edge_index[0], N + (jnp.arange(npad, dtype=jnp.int32) % (NP - N))])
    col = jnp.concatenate([edge_index[1], jnp.zeros(npad, dtype=jnp.int32)])
    x = jnp.pad(local_preds, ((0, NP - N), (0, 0)))
    ones8 = jnp.ones((CH, HC), dtype=jnp.float32)
    zeros8 = jnp.zeros((RPT, HC), dtype=jnp.float32)

    col2 = col.reshape(NS * NCHUNK, CH)
    row2 = row.reshape(NS * NCHUNK, CH)

    loc = _dense_stage(x, W1, W2)
    loc2 = jnp.stack([loc[:, :HC], loc[:, HC:]])

    out = _prop_kernel(loc2, col2, row2, ones8, zeros8)
    preds = jnp.concatenate([out[0], out[1]], axis=1)
    return preds[:N]


# speedup vs baseline: 1.0312x; 1.0312x over previous
import jax
import jax.numpy as jnp
from jax import lax
from jax.experimental import pallas as pl
from jax.experimental.pallas import tpu as pltpu
from jax.experimental.pallas import tpu_sc as plsc

N = 10000
E = 320000
IN_FEATS = 128
N_HIDDEN = 64
C = 16
ALPHA = 0.1
NITER = 10

NS = 16
NP = 10240
RPT = NP // NS
EP = 327680
EPT = EP // NS
CH = 1024
NCHUNK = EPT // CH

HC = C // 2
NPH = NP // 2
RPT2 = NPH // NS

_MESH1 = dict(
    mesh=plsc.VectorSubcoreMesh(
        core_axis_name="c", subcore_axis_name="s", num_cores=1, num_subcores=NS
    ),
    compiler_params=pltpu.CompilerParams(
        use_tc_tiling_on_sc=False, needs_layout_passes=False),
)
_MESH2 = dict(
    mesh=plsc.VectorSubcoreMesh(
        core_axis_name="c", subcore_axis_name="s", num_cores=2, num_subcores=NS
    ),
    compiler_params=pltpu.CompilerParams(
        use_tc_tiling_on_sc=False, needs_layout_passes=False),
)


def _degree_body(row2_hbm, ones_hbm, zeros_hbm, deg_out,
                 acc_sh, cidx_a, ones_v, sem):
    cix = lax.axis_index("c")
    w = lax.axis_index("s")
    rsl = pl.ds(pl.multiple_of(w * RPT, RPT), RPT)
    pltpu.sync_copy(row2_hbm.at[pl.ds(w * NCHUNK, NCHUNK)], cidx_a)
    pltpu.sync_copy(ones_hbm, ones_v)
    pltpu.sync_copy(zeros_hbm, acc_sh.at[rsl])
    plsc.subcore_barrier()
    cps = [pltpu.async_copy(ones_v, acc_sh.at[cidx_a.at[j]], sem, add=True)
           for j in range(NCHUNK)]
    for cp in cps:
        cp.wait()
    plsc.subcore_barrier()
    pltpu.sync_copy(acc_sh.at[rsl], deg_out.at[cix, rsl])


_BLK = 1024


def _dense_body(x_ref, w1_ref, w2_ref, loc_ref):
    h = jnp.tanh(jnp.dot(x_ref[...], w1_ref[...],
                         preferred_element_type=jnp.float32))
    loc_ref[...] = jnp.dot(h, w2_ref[...], preferred_element_type=jnp.float32)


def _dense_stage(x, w1, w2):
    return pl.pallas_call(
        _dense_body,
        grid=(NP // _BLK,),
        in_specs=[
            pl.BlockSpec((_BLK, IN_FEATS), lambda i: (i, 0)),
            pl.BlockSpec((IN_FEATS, N_HIDDEN), lambda i: (0, 0)),
            pl.BlockSpec((N_HIDDEN, C), lambda i: (0, 0)),
        ],
        out_specs=pl.BlockSpec((_BLK, C), lambda i: (i, 0)),
        out_shape=jax.ShapeDtypeStruct((NP, C), jnp.float32),
    )(x, w1, w2)


NBUF = 4


def _prop_body(deg2_hbm, loc2_hbm, col2_hbm, row2_hbm,
               out_hbm,
               q_sh, acc_sh, cidx_a, ridx_a, msg0, msg1, msg2, msg3,
               a_v, b_v, sa_v, sb_v, sal_v, sbl_v,
               gs0, gs1, gs2, gs3, ss0, ss1, ss2, ss3):
    cix = lax.axis_index("c")
    w = lax.axis_index("s")
    rsl = pl.ds(pl.multiple_of(w * RPT, RPT), RPT)
    pltpu.sync_copy(col2_hbm.at[pl.ds(w * NCHUNK, NCHUNK)], cidx_a)
    pltpu.sync_copy(row2_hbm.at[pl.ds(w * NCHUNK, NCHUNK)], ridx_a)
    lane = lax.iota(jnp.int32, 16)
    r0 = lax.shift_right_logical(lane, 3)
    c16 = lax.bitwise_and(lane, 7)

    pltpu.sync_copy(deg2_hbm.at[cix, rsl], a_v)
    pltpu.sync_copy(loc2_hbm.at[cix, rsl], b_v)

    def prep(i, carry):
        ri = r0 + 2 * i
        d = plsc.load_gather(a_v, [ri, c16]) + 1.0
        lo = plsc.load_gather(b_v, [ri, c16])
        ii = 0x5F3759DF - lax.shift_right_logical(
            plsc.bitcast(d, jnp.int32), 1)
        y = plsc.bitcast(ii, jnp.float32)
        for _ in range(4):
            y = y * (1.5 - 0.5 * d * y * y)
        q0 = y * lo
        plsc.store_scatter(sa_v, [ri, c16], 0.9 * y * y)
        plsc.store_scatter(sal_v, [ri, c16], 0.9 * y)
        plsc.store_scatter(sb_v, [ri, c16], ALPHA * q0)
        plsc.store_scatter(sbl_v, [ri, c16], ALPHA * lo)
        plsc.store_scatter(a_v, [ri, c16], q0)
        return carry

    lax.fori_loop(0, RPT // 2, prep, 0)
    bufs = (q_sh, acc_sh)
    pltpu.sync_copy(a_v, q_sh.at[rsl])
    pltpu.sync_copy(a_v, acc_sh.at[rsl])
    plsc.subcore_barrier()

    msgs = (msg0, msg1, msg2, msg3)
    gsems = (gs0, gs1, gs2, gs3)
    ssems = (ss0, ss1, ss2, ss3)

    for t in range(NITER):
        S = bufs[t % 2]
        ACC = bufs[(t + 1) % 2]
        s1_v = sa_v if t < NITER - 1 else sal_v
        s2_v = sb_v if t < NITER - 1 else sbl_v

        gcp = [None] * NCHUNK
        scp = [None] * NCHUNK
        for p in range(min(3, NCHUNK)):
            gcp[p] = pltpu.async_copy(S.at[cidx_a.at[p]], msgs[p], gsems[p])
        for j in range(NCHUNK):
            b = j % NBUF
            gcp[j].wait()
            scp[j] = pltpu.async_copy(
                msgs[b], ACC.at[ridx_a.at[j]], ssems[b], add=True)
            nxt = j + 3
            if nxt < NCHUNK:
                nb = nxt % NBUF
                if nxt >= NBUF:
                    scp[nxt - NBUF].wait()
                gcp[nxt] = pltpu.async_copy(
                    S.at[cidx_a.at[nxt]], msgs[nb], gsems[nb])
        for j in range(max(0, NCHUNK - NBUF), NCHUNK):
            scp[j].wait()
        plsc.subcore_barrier()

        pltpu.sync_copy(ACC.at[rsl], a_v)

        def row(i, carry):
            ri = r0 + 2 * i
            a = plsc.load_gather(a_v, [ri, c16])
            bb = plsc.load_gather(s1_v, [ri, c16])
            cc = plsc.load_gather(s2_v, [ri, c16])
            plsc.store_scatter(a_v, [ri, c16], a * bb + cc)
            return carry

        lax.fori_loop(0, RPT // 2, row, 0)
        if t < NITER - 1:
            pltpu.sync_copy(a_v, ACC.at[rsl])
            pltpu.sync_copy(a_v, S.at[rsl])
        else:
            pltpu.sync_copy(a_v, out_hbm.at[cix, rsl])
        plsc.subcore_barrier()


def _build(interpret=False):
    degree_kernel = pl.kernel(
        _degree_body,
        out_type=jax.ShapeDtypeStruct((2, NP, HC), jnp.float32),
        scratch_types=[
            pltpu.VMEM_SHARED((NP, HC), jnp.float32),
            pltpu.VMEM((NCHUNK, CH), jnp.int32),
            pltpu.VMEM((CH, HC), jnp.float32),
            pltpu.SemaphoreType.DMA,
        ],
        interpret=interpret,
        **_MESH2,
    )
    prop_kernel = pl.kernel(
        _prop_body,
        out_type=jax.ShapeDtypeStruct((2, NP, HC), jnp.float32),
        scratch_types=[
            pltpu.VMEM_SHARED((NP, HC), jnp.float32),
            pltpu.VMEM_SHARED((NP, HC), jnp.float32),
            pltpu.VMEM((NCHUNK, CH), jnp.int32),
            pltpu.VMEM((NCHUNK, CH), jnp.int32),
            pltpu.VMEM((CH, HC), jnp.float32),
            pltpu.VMEM((CH, HC), jnp.float32),
            pltpu.VMEM((CH, HC), jnp.float32),
            pltpu.VMEM((CH, HC), jnp.float32),
            pltpu.VMEM((RPT, HC), jnp.float32),
            pltpu.VMEM((RPT, HC), jnp.float32),
            pltpu.VMEM((RPT, HC), jnp.float32),
            pltpu.VMEM((RPT, HC), jnp.float32),
            pltpu.VMEM((RPT, HC), jnp.float32),
            pltpu.VMEM((RPT, HC), jnp.float32),
        ] + [pltpu.SemaphoreType.DMA] * 8,
        interpret=interpret,
        **_MESH2,
    )
    return degree_kernel, prop_kernel


_degree_kernel, _prop_kernel = _build()


def kernel(local_preds, edge_index, W1, W2):
    npad = EP - E
    row = jnp.concatenate(
        [edge_index[0], N + (jnp.arange(npad, dtype=jnp.int32) % (NP - N))])
    col = jnp.concatenate([edge_index[1], jnp.zeros(npad, dtype=jnp.int32)])
    x = jnp.pad(local_preds, ((0, NP - N), (0, 0)))
    ones8 = jnp.ones((CH, HC), dtype=jnp.float32)
    zeros8 = jnp.zeros((RPT, HC), dtype=jnp.float32)

    col2 = col.reshape(NS * NCHUNK, CH)
    row2 = row.reshape(NS * NCHUNK, CH)

    deg2 = _degree_kernel(row2, ones8, zeros8)
    loc = _dense_stage(x, W1, W2)
    loc2 = jnp.stack([loc[:, :HC], loc[:, HC:]])

    out = _prop_kernel(deg2, loc2, col2, row2)
    preds = jnp.concatenate([out[0], out[1]], axis=1)
    return preds[:N]


# speedup vs baseline: 1.0494x; 1.0176x over previous
import jax
import jax.numpy as jnp
from jax import lax
from jax.experimental import pallas as pl
from jax.experimental.pallas import tpu as pltpu
from jax.experimental.pallas import tpu_sc as plsc

N = 10000
E = 320000
IN_FEATS = 128
N_HIDDEN = 64
C = 16
ALPHA = 0.1
NITER = 10

NS = 16
NP = 10240
RPT = NP // NS
EP = 327680
EPT = EP // NS
CH = 1024
NCHUNK = EPT // CH

HC = C // 2
NPH = NP // 2
RPT2 = NPH // NS

_MESH1 = dict(
    mesh=plsc.VectorSubcoreMesh(
        core_axis_name="c", subcore_axis_name="s", num_cores=1, num_subcores=NS
    ),
    compiler_params=pltpu.CompilerParams(
        use_tc_tiling_on_sc=False, needs_layout_passes=False),
)
_MESH2 = dict(
    mesh=plsc.VectorSubcoreMesh(
        core_axis_name="c", subcore_axis_name="s", num_cores=2, num_subcores=NS
    ),
    compiler_params=pltpu.CompilerParams(
        use_tc_tiling_on_sc=False, needs_layout_passes=False),
)


def _degree_body(row2_hbm, ones_hbm, zeros_hbm, deg_out,
                 acc_sh, cidx_a, ones_v, sem):
    cix = lax.axis_index("c")
    w = lax.axis_index("s")
    rsl = pl.ds(pl.multiple_of(w * RPT, RPT), RPT)
    pltpu.sync_copy(row2_hbm.at[pl.ds(w * NCHUNK, NCHUNK)], cidx_a)
    pltpu.sync_copy(ones_hbm, ones_v)
    pltpu.sync_copy(zeros_hbm, acc_sh.at[rsl])
    plsc.subcore_barrier()
    cps = [pltpu.async_copy(ones_v, acc_sh.at[cidx_a.at[j]], sem, add=True)
           for j in range(NCHUNK)]
    for cp in cps:
        cp.wait()
    plsc.subcore_barrier()
    pltpu.sync_copy(acc_sh.at[rsl], deg_out.at[cix, rsl])


_BLK = 1024


def _dense_body(x_ref, w1_ref, w2_ref, la_ref, lb_ref):
    h = jnp.tanh(jnp.dot(x_ref[...], w1_ref[...],
                         preferred_element_type=jnp.float32))
    loc = jnp.dot(h, w2_ref[...], preferred_element_type=jnp.float32)
    la_ref[...] = loc[:, :HC]
    lb_ref[...] = loc[:, HC:]


def _dense_stage(x, w1, w2):
    return pl.pallas_call(
        _dense_body,
        grid=(NP // _BLK,),
        in_specs=[
            pl.BlockSpec((_BLK, IN_FEATS), lambda i: (i, 0)),
            pl.BlockSpec((IN_FEATS, N_HIDDEN), lambda i: (0, 0)),
            pl.BlockSpec((N_HIDDEN, C), lambda i: (0, 0)),
        ],
        out_specs=[pl.BlockSpec((_BLK, HC), lambda i: (i, 0))] * 2,
        out_shape=[jax.ShapeDtypeStruct((NP, HC), jnp.float32)] * 2,
    )(x, w1, w2)


NBUF = 4


def _prop_body(deg2_hbm, loca_hbm, locb_hbm, col2_hbm, row2_hbm,
               out_hbm,
               q_sh, acc_sh, cidx_a, ridx_a, msg0, msg1, msg2, msg3,
               a_v, b_v, sa_v, sb_v, sal_v, sbl_v,
               gs0, gs1, gs2, gs3, ss0, ss1, ss2, ss3):
    cix = lax.axis_index("c")
    w = lax.axis_index("s")
    rsl = pl.ds(pl.multiple_of(w * RPT, RPT), RPT)
    pltpu.sync_copy(col2_hbm.at[pl.ds(w * NCHUNK, NCHUNK)], cidx_a)
    pltpu.sync_copy(row2_hbm.at[pl.ds(w * NCHUNK, NCHUNK)], ridx_a)
    lane = lax.iota(jnp.int32, 16)
    r0 = lax.shift_right_logical(lane, 3)
    c16 = lax.bitwise_and(lane, 7)

    pltpu.sync_copy(deg2_hbm.at[cix, rsl], a_v)

    @pl.when(cix == 0)
    def _():
        pltpu.sync_copy(loca_hbm.at[rsl], b_v)

    @pl.when(cix == 1)
    def _():
        pltpu.sync_copy(locb_hbm.at[rsl], b_v)

    def prep(i, carry):
        ri = r0 + 2 * i
        d = plsc.load_gather(a_v, [ri, c16]) + 1.0
        lo = plsc.load_gather(b_v, [ri, c16])
        ii = 0x5F3759DF - lax.shift_right_logical(
            plsc.bitcast(d, jnp.int32), 1)
        y = plsc.bitcast(ii, jnp.float32)
        for _ in range(4):
            y = y * (1.5 - 0.5 * d * y * y)
        q0 = y * lo
        plsc.store_scatter(sa_v, [ri, c16], 0.9 * y * y)
        plsc.store_scatter(sal_v, [ri, c16], 0.9 * y)
        plsc.store_scatter(sb_v, [ri, c16], ALPHA * q0)
        plsc.store_scatter(sbl_v, [ri, c16], ALPHA * lo)
        plsc.store_scatter(a_v, [ri, c16], q0)
        return carry

    lax.fori_loop(0, RPT // 2, prep, 0)
    bufs = (q_sh, acc_sh)
    pltpu.sync_copy(a_v, q_sh.at[rsl])
    pltpu.sync_copy(a_v, acc_sh.at[rsl])
    plsc.subcore_barrier()

    msgs = (msg0, msg1, msg2, msg3)
    gsems = (gs0, gs1, gs2, gs3)
    ssems = (ss0, ss1, ss2, ss3)

    for t in range(NITER):
        S = bufs[t % 2]
        ACC = bufs[(t + 1) % 2]
        s1_v = sa_v if t < NITER - 1 else sal_v
        s2_v = sb_v if t < NITER - 1 else sbl_v

        gcp = [None] * NCHUNK
        scp = [None] * NCHUNK
        for p in range(min(3, NCHUNK)):
            gcp[p] = pltpu.async_copy(S.at[cidx_a.at[p]], msgs[p], gsems[p])
        for j in range(NCHUNK):
            b = j % NBUF
            gcp[j].wait()
            scp[j] = pltpu.async_copy(
                msgs[b], ACC.at[ridx_a.at[j]], ssems[b], add=True)
            nxt = j + 3
            if nxt < NCHUNK:
                nb = nxt % NBUF
                if nxt >= NBUF:
                    scp[nxt - NBUF].wait()
                gcp[nxt] = pltpu.async_copy(
                    S.at[cidx_a.at[nxt]], msgs[nb], gsems[nb])
        for j in range(max(0, NCHUNK - NBUF), NCHUNK):
            scp[j].wait()
        plsc.subcore_barrier()

        pltpu.sync_copy(ACC.at[rsl], a_v)

        def row(i, carry):
            ri = r0 + 2 * i
            a = plsc.load_gather(a_v, [ri, c16])
            bb = plsc.load_gather(s1_v, [ri, c16])
            cc = plsc.load_gather(s2_v, [ri, c16])
            plsc.store_scatter(a_v, [ri, c16], a * bb + cc)
            return carry

        lax.fori_loop(0, RPT // 2, row, 0)
        if t < NITER - 1:
            pltpu.sync_copy(a_v, ACC.at[rsl])
            pltpu.sync_copy(a_v, S.at[rsl])
        else:
            pltpu.sync_copy(a_v, out_hbm.at[cix, rsl])
        plsc.subcore_barrier()


def _build(interpret=False):
    degree_kernel = pl.kernel(
        _degree_body,
        out_type=jax.ShapeDtypeStruct((2, NP, HC), jnp.float32),
        scratch_types=[
            pltpu.VMEM_SHARED((NP, HC), jnp.float32),
            pltpu.VMEM((NCHUNK, CH), jnp.int32),
            pltpu.VMEM((CH, HC), jnp.float32),
            pltpu.SemaphoreType.DMA,
        ],
        interpret=interpret,
        **_MESH2,
    )
    prop_kernel = pl.kernel(
        _prop_body,
        out_type=jax.ShapeDtypeStruct((2, NP, HC), jnp.float32),
        scratch_types=[
            pltpu.VMEM_SHARED((NP, HC), jnp.float32),
            pltpu.VMEM_SHARED((NP, HC), jnp.float32),
            pltpu.VMEM((NCHUNK, CH), jnp.int32),
            pltpu.VMEM((NCHUNK, CH), jnp.int32),
            pltpu.VMEM((CH, HC), jnp.float32),
            pltpu.VMEM((CH, HC), jnp.float32),
            pltpu.VMEM((CH, HC), jnp.float32),
            pltpu.VMEM((CH, HC), jnp.float32),
            pltpu.VMEM((RPT, HC), jnp.float32),
            pltpu.VMEM((RPT, HC), jnp.float32),
            pltpu.VMEM((RPT, HC), jnp.float32),
            pltpu.VMEM((RPT, HC), jnp.float32),
            pltpu.VMEM((RPT, HC), jnp.float32),
            pltpu.VMEM((RPT, HC), jnp.float32),
        ] + [pltpu.SemaphoreType.DMA] * 8,
        interpret=interpret,
        **_MESH2,
    )
    return degree_kernel, prop_kernel


_degree_kernel, _prop_kernel = _build()


def kernel(local_preds, edge_index, W1, W2):
    npad = EP - E
    row = jnp.concatenate(
        [edge_index[0], N + (jnp.arange(npad, dtype=jnp.int32) % (NP - N))])
    col = jnp.concatenate([edge_index[1], jnp.zeros(npad, dtype=jnp.int32)])
    x = jnp.pad(local_preds, ((0, NP - N), (0, 0)))
    ones8 = jnp.ones((CH, HC), dtype=jnp.float32)
    zeros8 = jnp.zeros((RPT, HC), dtype=jnp.float32)

    col2 = col.reshape(NS * NCHUNK, CH)
    row2 = row.reshape(NS * NCHUNK, CH)

    deg2 = _degree_kernel(row2, ones8, zeros8)
    loca, locb = _dense_stage(x, W1, W2)

    out = _prop_kernel(deg2, loca, locb, col2, row2)
    preds = jnp.concatenate([out[0], out[1]], axis=1)
    return preds[:N]


# speedup vs baseline: 1.0943x; 1.0429x over previous
import jax
import jax.numpy as jnp
from jax import lax
from jax.experimental import pallas as pl
from jax.experimental.pallas import tpu as pltpu
from jax.experimental.pallas import tpu_sc as plsc

N = 10000
E = 320000
IN_FEATS = 128
N_HIDDEN = 64
C = 16
ALPHA = 0.1
NITER = 10

NS = 16
NP = 10240
RPT = NP // NS
EP = 327680
EPT = EP // NS
CH = 1024
NCHUNK = EPT // CH

HC = C // 2
NPH = NP // 2
RPT2 = NPH // NS

_MESH1 = dict(
    mesh=plsc.VectorSubcoreMesh(
        core_axis_name="c", subcore_axis_name="s", num_cores=1, num_subcores=NS
    ),
    compiler_params=pltpu.CompilerParams(
        use_tc_tiling_on_sc=False, needs_layout_passes=False),
)
_MESH2 = dict(
    mesh=plsc.VectorSubcoreMesh(
        core_axis_name="c", subcore_axis_name="s", num_cores=2, num_subcores=NS
    ),
    compiler_params=pltpu.CompilerParams(
        use_tc_tiling_on_sc=False, needs_layout_passes=False),
)


def _degree_body(row2_hbm, ones_hbm, zeros_hbm, deg_out,
                 acc_sh, cidx_a, ones_v, sem):
    cix = lax.axis_index("c")
    w = lax.axis_index("s")
    rsl = pl.ds(pl.multiple_of(w * RPT, RPT), RPT)
    pltpu.sync_copy(row2_hbm.at[pl.ds(w * NCHUNK, NCHUNK)], cidx_a)
    pltpu.sync_copy(ones_hbm, ones_v)
    pltpu.sync_copy(zeros_hbm, acc_sh.at[rsl])
    plsc.subcore_barrier()
    cps = [pltpu.async_copy(ones_v, acc_sh.at[cidx_a.at[j]], sem, add=True)
           for j in range(NCHUNK)]
    for cp in cps:
        cp.wait()
    plsc.subcore_barrier()
    pltpu.sync_copy(acc_sh.at[rsl], deg_out.at[cix, rsl])


_BLK = 1024


def _dense_body(x_ref, w1_ref, w2_ref, la_ref, lb_ref):
    h = jnp.tanh(jnp.dot(x_ref[...], w1_ref[...],
                         preferred_element_type=jnp.float32))
    loc = jnp.dot(h, w2_ref[...], preferred_element_type=jnp.float32)
    la_ref[...] = loc[:, :HC]
    lb_ref[...] = loc[:, HC:]


def _dense_stage(x, w1, w2):
    return pl.pallas_call(
        _dense_body,
        grid=(NP // _BLK,),
        in_specs=[
            pl.BlockSpec((_BLK, IN_FEATS), lambda i: (i, 0)),
            pl.BlockSpec((IN_FEATS, N_HIDDEN), lambda i: (0, 0)),
            pl.BlockSpec((N_HIDDEN, C), lambda i: (0, 0)),
        ],
        out_specs=[pl.BlockSpec((_BLK, HC), lambda i: (i, 0))] * 2,
        out_shape=[jax.ShapeDtypeStruct((NP, HC), jnp.float32)] * 2,
    )(x, w1, w2)


NBUF = 4


def _prop_body(deg2_hbm, loca_hbm, locb_hbm, col2_hbm, row2_hbm,
               out_hbm,
               q_sh, acc_sh, cidx_a, ridx_a, msg0, msg1, msg2, msg3,
               a_v, b_v, sa_v, sb_v, sal_v, sbl_v,
               gs0, gs1, gs2, gs3, ss0, ss1, ss2, ss3):
    cix = lax.axis_index("c")
    w = lax.axis_index("s")
    rsl = pl.ds(pl.multiple_of(w * RPT, RPT), RPT)
    pltpu.sync_copy(col2_hbm.at[pl.ds(w * NCHUNK, NCHUNK)], cidx_a)
    pltpu.sync_copy(row2_hbm.at[pl.ds(w * NCHUNK, NCHUNK)], ridx_a)
    lane = lax.iota(jnp.int32, 16)
    r0 = lax.shift_right_logical(lane, 3)
    c16 = lax.bitwise_and(lane, 7)

    pltpu.sync_copy(deg2_hbm.at[cix, rsl], a_v)

    @pl.when(cix == 0)
    def _():
        pltpu.sync_copy(loca_hbm.at[rsl], b_v)

    @pl.when(cix == 1)
    def _():
        pltpu.sync_copy(locb_hbm.at[rsl], b_v)

    def prep(i, carry):
        ri = r0 + 2 * i
        d = plsc.load_gather(a_v, [ri, c16]) + 1.0
        lo = plsc.load_gather(b_v, [ri, c16])
        ii = 0x5F3759DF - lax.shift_right_logical(
            plsc.bitcast(d, jnp.int32), 1)
        y = plsc.bitcast(ii, jnp.float32)
        for _ in range(4):
            y = y * (1.5 - 0.5 * d * y * y)
        q0 = y * lo
        plsc.store_scatter(sa_v, [ri, c16], 0.9 * y * y)
        plsc.store_scatter(sal_v, [ri, c16], 0.9 * y)
        plsc.store_scatter(sb_v, [ri, c16], ALPHA * q0)
        plsc.store_scatter(sbl_v, [ri, c16], ALPHA * lo)
        plsc.store_scatter(a_v, [ri, c16], q0)
        return carry

    lax.fori_loop(0, RPT // 2, prep, 0)
    bufs = (q_sh, acc_sh)
    pltpu.sync_copy(a_v, q_sh.at[rsl])
    pltpu.sync_copy(a_v, acc_sh.at[rsl])
    plsc.subcore_barrier()

    msgs = (msg0, msg1, msg2, msg3)
    gsems = (gs0, gs1, gs2, gs3)
    ssems = (ss0, ss1, ss2, ss3)

    for t in range(NITER):
        S = bufs[t % 2]
        ACC = bufs[(t + 1) % 2]
        s1_v = sa_v if t < NITER - 1 else sal_v
        s2_v = sb_v if t < NITER - 1 else sbl_v

        gcp = [None] * NCHUNK
        scp = [None] * NCHUNK
        for p in range(min(3, NCHUNK)):
            gcp[p] = pltpu.async_copy(S.at[cidx_a.at[p]], msgs[p], gsems[p])
        for j in range(NCHUNK):
            b = j % NBUF
            gcp[j].wait()
            scp[j] = pltpu.async_copy(
                msgs[b], ACC.at[ridx_a.at[j]], ssems[b], add=True)
            nxt = j + 3
            if nxt < NCHUNK:
                nb = nxt % NBUF
                if nxt >= NBUF:
                    scp[nxt - NBUF].wait()
                gcp[nxt] = pltpu.async_copy(
                    S.at[cidx_a.at[nxt]], msgs[nb], gsems[nb])
        for j in range(max(0, NCHUNK - NBUF), NCHUNK):
            scp[j].wait()
        plsc.subcore_barrier()

        pltpu.sync_copy(ACC.at[rsl], a_v)

        def row(i, carry):
            ri = r0 + 2 * i
            a = plsc.load_gather(a_v, [ri, c16])
            bb = plsc.load_gather(s1_v, [ri, c16])
            cc = plsc.load_gather(s2_v, [ri, c16])
            plsc.store_scatter(a_v, [ri, c16], a * bb + cc)
            return carry

        lax.fori_loop(0, RPT // 2, row, 0)
        if t < NITER - 1:
            pltpu.sync_copy(a_v, ACC.at[rsl])
            pltpu.sync_copy(a_v, S.at[rsl])
        else:
            @pl.when(cix == 0)
            def _():
                pltpu.sync_copy(a_v, out_hbm.at[rsl, pl.ds(0, HC)])

            @pl.when(cix == 1)
            def _():
                pltpu.sync_copy(a_v, out_hbm.at[rsl, pl.ds(HC, HC)])
        plsc.subcore_barrier()


def _build(interpret=False):
    degree_kernel = pl.kernel(
        _degree_body,
        out_type=jax.ShapeDtypeStruct((2, NP, HC), jnp.float32),
        scratch_types=[
            pltpu.VMEM_SHARED((NP, HC), jnp.float32),
            pltpu.VMEM((NCHUNK, CH), jnp.int32),
            pltpu.VMEM((CH, HC), jnp.float32),
            pltpu.SemaphoreType.DMA,
        ],
        interpret=interpret,
        **_MESH2,
    )
    prop_kernel = pl.kernel(
        _prop_body,
        out_type=jax.ShapeDtypeStruct((NP, C), jnp.float32),
        scratch_types=[
            pltpu.VMEM_SHARED((NP, HC), jnp.float32),
            pltpu.VMEM_SHARED((NP, HC), jnp.float32),
            pltpu.VMEM((NCHUNK, CH), jnp.int32),
            pltpu.VMEM((NCHUNK, CH), jnp.int32),
            pltpu.VMEM((CH, HC), jnp.float32),
            pltpu.VMEM((CH, HC), jnp.float32),
            pltpu.VMEM((CH, HC), jnp.float32),
            pltpu.VMEM((CH, HC), jnp.float32),
            pltpu.VMEM((RPT, HC), jnp.float32),
            pltpu.VMEM((RPT, HC), jnp.float32),
            pltpu.VMEM((RPT, HC), jnp.float32),
            pltpu.VMEM((RPT, HC), jnp.float32),
            pltpu.VMEM((RPT, HC), jnp.float32),
            pltpu.VMEM((RPT, HC), jnp.float32),
        ] + [pltpu.SemaphoreType.DMA] * 8,
        interpret=interpret,
        **_MESH2,
    )
    return degree_kernel, prop_kernel


_degree_kernel, _prop_kernel = _build()


def kernel(local_preds, edge_index, W1, W2):
    npad = EP - E
    row = jnp.concatenate(
        [edge_index[0], N + (jnp.arange(npad, dtype=jnp.int32) % (NP - N))])
    col = jnp.concatenate([edge_index[1], jnp.zeros(npad, dtype=jnp.int32)])
    x = jnp.pad(local_preds, ((0, NP - N), (0, 0)))
    ones8 = jnp.ones((CH, HC), dtype=jnp.float32)
    zeros8 = jnp.zeros((RPT, HC), dtype=jnp.float32)

    col2 = col.reshape(NS * NCHUNK, CH)
    row2 = row.reshape(NS * NCHUNK, CH)

    deg2 = _degree_kernel(row2, ones8, zeros8)
    loca, locb = _dense_stage(x, W1, W2)

    preds = _prop_kernel(deg2, loca, locb, col2, row2)
    return preds[:N]


# speedup vs baseline: 1.0946x; 1.0002x over previous
import jax
import jax.numpy as jnp
from jax import lax
from jax.experimental import pallas as pl
from jax.experimental.pallas import tpu as pltpu
from jax.experimental.pallas import tpu_sc as plsc

N = 10000
E = 320000
IN_FEATS = 128
N_HIDDEN = 64
C = 16
ALPHA = 0.1
NITER = 10

NS = 16
NP = 10240
RPT = NP // NS
EP = 327680
EPT = EP // NS
CH = 1024
NCHUNK = EPT // CH

HC = C // 2
_MESH2 = dict(
    mesh=plsc.VectorSubcoreMesh(
        core_axis_name="c", subcore_axis_name="s", num_cores=2, num_subcores=NS
    ),
    compiler_params=pltpu.CompilerParams(
        use_tc_tiling_on_sc=False, needs_layout_passes=False),
)


def _degree_body(row2_hbm, ones_hbm, zeros_hbm, deg_out,
                 acc_sh, cidx_a, ones_v, sem):
    cix = lax.axis_index("c")
    w = lax.axis_index("s")
    rsl = pl.ds(pl.multiple_of(w * RPT, RPT), RPT)
    pltpu.sync_copy(row2_hbm.at[pl.ds(w * NCHUNK, NCHUNK)], cidx_a)
    pltpu.sync_copy(ones_hbm, ones_v)
    pltpu.sync_copy(zeros_hbm, acc_sh.at[rsl])
    plsc.subcore_barrier()
    cps = [pltpu.async_copy(ones_v, acc_sh.at[cidx_a.at[j]], sem, add=True)
           for j in range(NCHUNK)]
    for cp in cps:
        cp.wait()
    plsc.subcore_barrier()
    pltpu.sync_copy(acc_sh.at[rsl], deg_out.at[cix, rsl])


_BLK = 1024


def _dense_body(x_ref, w1_ref, w2_ref, la_ref, lb_ref):
    h = jnp.tanh(jnp.dot(x_ref[...], w1_ref[...],
                         preferred_element_type=jnp.float32))
    loc = jnp.dot(h, w2_ref[...], preferred_element_type=jnp.float32)
    la_ref[...] = loc[:, :HC]
    lb_ref[...] = loc[:, HC:]


def _dense_stage(x, w1, w2):
    return pl.pallas_call(
        _dense_body,
        grid=(NP // _BLK,),
        in_specs=[
            pl.BlockSpec((_BLK, IN_FEATS), lambda i: (i, 0)),
            pl.BlockSpec((IN_FEATS, N_HIDDEN), lambda i: (0, 0)),
            pl.BlockSpec((N_HIDDEN, C), lambda i: (0, 0)),
        ],
        out_specs=[pl.BlockSpec((_BLK, HC), lambda i: (i, 0))] * 2,
        out_shape=[jax.ShapeDtypeStruct((NP, HC), jnp.float32)] * 2,
    )(x, w1, w2)


NBUF = 4


def _prop_body(deg2_hbm, loca_hbm, locb_hbm, col2_hbm, row2_hbm,
               out_hbm,
               q_sh, acc_sh, cidx_a, ridx_a, msg0, msg1, msg2, msg3,
               a_v, b_v, sa_v, sb_v, sal_v, sbl_v,
               gs0, gs1, gs2, gs3, ss0, ss1, ss2, ss3):
    cix = lax.axis_index("c")
    w = lax.axis_index("s")
    rsl = pl.ds(pl.multiple_of(w * RPT, RPT), RPT)
    pltpu.sync_copy(col2_hbm.at[pl.ds(w * NCHUNK, NCHUNK)], cidx_a)
    pltpu.sync_copy(row2_hbm.at[pl.ds(w * NCHUNK, NCHUNK)], ridx_a)
    lane = lax.iota(jnp.int32, 16)
    r0 = lax.shift_right_logical(lane, 3)
    c16 = lax.bitwise_and(lane, 7)

    pltpu.sync_copy(deg2_hbm.at[cix, rsl], a_v)

    @pl.when(cix == 0)
    def _():
        pltpu.sync_copy(loca_hbm.at[rsl], b_v)

    @pl.when(cix == 1)
    def _():
        pltpu.sync_copy(locb_hbm.at[rsl], b_v)

    def prep(i, carry):
        ri = r0 + 2 * i
        d = plsc.load_gather(a_v, [ri, c16]) + 1.0
        lo = plsc.load_gather(b_v, [ri, c16])
        ii = 0x5F3759DF - lax.shift_right_logical(
            plsc.bitcast(d, jnp.int32), 1)
        y = plsc.bitcast(ii, jnp.float32)
        for _ in range(4):
            y = y * (1.5 - 0.5 * d * y * y)
        q0 = y * lo
        plsc.store_scatter(sa_v, [ri, c16], 0.9 * y * y)
        plsc.store_scatter(sal_v, [ri, c16], 0.9 * y)
        plsc.store_scatter(sb_v, [ri, c16], ALPHA * q0)
        plsc.store_scatter(sbl_v, [ri, c16], ALPHA * lo)
        plsc.store_scatter(a_v, [ri, c16], q0)
        return carry

    lax.fori_loop(0, RPT // 2, prep, 0)
    bufs = (q_sh, acc_sh)
    pltpu.sync_copy(a_v, q_sh.at[rsl])
    pltpu.sync_copy(a_v, acc_sh.at[rsl])
    plsc.subcore_barrier()

    msgs = (msg0, msg1, msg2, msg3)
    gsems = (gs0, gs1, gs2, gs3)
    ssems = (ss0, ss1, ss2, ss3)

    for t in range(NITER):
        S = bufs[t % 2]
        ACC = bufs[(t + 1) % 2]
        s1_v = sa_v if t < NITER - 1 else sal_v
        s2_v = sb_v if t < NITER - 1 else sbl_v

        gcp = [None] * NCHUNK
        scp = [None] * NCHUNK
        for p in range(min(3, NCHUNK)):
            gcp[p] = pltpu.async_copy(S.at[cidx_a.at[p]], msgs[p], gsems[p])
        for j in range(NCHUNK):
            b = j % NBUF
            gcp[j].wait()
            scp[j] = pltpu.async_copy(
                msgs[b], ACC.at[ridx_a.at[j]], ssems[b], add=True)
            nxt = j + 3
            if nxt < NCHUNK:
                nb = nxt % NBUF
                if nxt >= NBUF:
                    scp[nxt - NBUF].wait()
                gcp[nxt] = pltpu.async_copy(
                    S.at[cidx_a.at[nxt]], msgs[nb], gsems[nb])
        for j in range(max(0, NCHUNK - NBUF), NCHUNK):
            scp[j].wait()
        plsc.subcore_barrier()

        pltpu.sync_copy(ACC.at[rsl], a_v)

        def row(i, carry):
            ri = r0 + 2 * i
            a = plsc.load_gather(a_v, [ri, c16])
            bb = plsc.load_gather(s1_v, [ri, c16])
            cc = plsc.load_gather(s2_v, [ri, c16])
            plsc.store_scatter(a_v, [ri, c16], a * bb + cc)
            return carry

        lax.fori_loop(0, RPT // 2, row, 0)
        if t < NITER - 1:
            pltpu.sync_copy(a_v, ACC.at[rsl])
            pltpu.sync_copy(a_v, S.at[rsl])
        else:
            @pl.when(cix == 0)
            def _():
                pltpu.sync_copy(a_v, out_hbm.at[rsl, pl.ds(0, HC)])

            @pl.when(cix == 1)
            def _():
                pltpu.sync_copy(a_v, out_hbm.at[rsl, pl.ds(HC, HC)])
        plsc.subcore_barrier()


def _build(interpret=False):
    degree_kernel = pl.kernel(
        _degree_body,
        out_type=jax.ShapeDtypeStruct((2, NP, HC), jnp.float32),
        scratch_types=[
            pltpu.VMEM_SHARED((NP, HC), jnp.float32),
            pltpu.VMEM((NCHUNK, CH), jnp.int32),
            pltpu.VMEM((CH, HC), jnp.float32),
            pltpu.SemaphoreType.DMA,
        ],
        interpret=interpret,
        **_MESH2,
    )
    prop_kernel = pl.kernel(
        _prop_body,
        out_type=jax.ShapeDtypeStruct((NP, C), jnp.float32),
        scratch_types=[
            pltpu.VMEM_SHARED((NP, HC), jnp.float32),
            pltpu.VMEM_SHARED((NP, HC), jnp.float32),
            pltpu.VMEM((NCHUNK, CH), jnp.int32),
            pltpu.VMEM((NCHUNK, CH), jnp.int32),
            pltpu.VMEM((CH, HC), jnp.float32),
            pltpu.VMEM((CH, HC), jnp.float32),
            pltpu.VMEM((CH, HC), jnp.float32),
            pltpu.VMEM((CH, HC), jnp.float32),
            pltpu.VMEM((RPT, HC), jnp.float32),
            pltpu.VMEM((RPT, HC), jnp.float32),
            pltpu.VMEM((RPT, HC), jnp.float32),
            pltpu.VMEM((RPT, HC), jnp.float32),
            pltpu.VMEM((RPT, HC), jnp.float32),
            pltpu.VMEM((RPT, HC), jnp.float32),
        ] + [pltpu.SemaphoreType.DMA] * 8,
        interpret=interpret,
        **_MESH2,
    )
    return degree_kernel, prop_kernel


_degree_kernel, _prop_kernel = _build()


def kernel(local_preds, edge_index, W1, W2):
    npad = EP - E
    row = jnp.concatenate(
        [edge_index[0], N + (jnp.arange(npad, dtype=jnp.int32) % (NP - N))])
    col = jnp.concatenate([edge_index[1], jnp.zeros(npad, dtype=jnp.int32)])
    x = jnp.pad(local_preds, ((0, NP - N), (0, 0)))
    ones8 = jnp.ones((CH, HC), dtype=jnp.float32)
    zeros8 = jnp.zeros((RPT, HC), dtype=jnp.float32)

    col2 = col.reshape(NS * NCHUNK, CH)
    row2 = row.reshape(NS * NCHUNK, CH)

    deg2 = _degree_kernel(row2, ones8, zeros8)
    loca, locb = _dense_stage(x, W1, W2)

    preds = _prop_kernel(deg2, loca, locb, col2, row2)
    return preds[:N]
